# Initial kernel scaffold; baseline (speedup 1.0000x reference)
import functools

import jax
import jax.numpy as jnp
from jax import lax
from jax.experimental import pallas as pl
from jax.experimental.pallas import tpu as pltpu
from jax.experimental.pallas import tpu_sc as plsc

N = 10000
D = 256
E = 160000

NC = 2
NS = 16
HALF = N // NC
NG = 1024
ACC_ROWS = 6400
W = 80
EPW = E // NS
NWIN = EPW // W
DRAIN_CH = 40
NDRAIN = HALF // DRAIN_CH
ZCH = 80

BR = 500


def _sc_segment_sum(h, src, dstl):
    mesh = plsc.VectorSubcoreMesh(core_axis_name="c", subcore_axis_name="s")

    @functools.partial(
        pl.kernel,
        out_type=jax.ShapeDtypeStruct((N, D), jnp.float32),
        mesh=mesh,
        scratch_types=[
            pltpu.VMEM((W,), jnp.int32),
            pltpu.VMEM((W,), jnp.int32),
            pltpu.VMEM((W, D), jnp.float32),
            pltpu.VMEM_SHARED((ACC_ROWS, D), jnp.float32),
        ],
    )
    def seg_sum_kernel(h_hbm, src_hbm, dstl_hbm, s_hbm, sidx, didx, stage, acc):
        c = lax.axis_index("c")
        s = lax.axis_index("s")

        @pl.loop(0, ZCH)
        def _(r):
            for j in range(D // 16):
                stage[r, pl.ds(j * 16, 16)] = jnp.zeros((16,), jnp.float32)

        zbase = s * (ACC_ROWS // NS)
        for z in range(ACC_ROWS // NS // ZCH):
            pltpu.sync_copy(stage.at[pl.ds(0, ZCH)],
                            acc.at[pl.ds(zbase + z * ZCH, ZCH)])
        plsc.subcore_barrier()

        ebase = s * EPW
        dbase = c * E + ebase

        @pl.loop(0, NWIN)
        def _(w):
            off = w * W
            pltpu.sync_copy(src_hbm.at[pl.ds(ebase + off, W)], sidx)
            pltpu.sync_copy(dstl_hbm.at[pl.ds(dbase + off, W)], didx)
            pltpu.sync_copy(h_hbm.at[sidx], stage)
            pltpu.sync_copy(stage, acc.at[didx], add=True)

        plsc.subcore_barrier()

        for k in range(pl.cdiv(NDRAIN, NS)):
            j = s + k * NS

            @pl.when(j < NDRAIN)
            def _():
                pltpu.sync_copy(
                    acc.at[pl.ds(j * DRAIN_CH, DRAIN_CH)],
                    s_hbm.at[pl.ds(c * HALF + j * DRAIN_CH, DRAIN_CH)],
                )

    return seg_sum_kernel(h, src, dstl)


def _mlp_body(eps_ref, h_ref, s_ref, w1_ref, b1_ref, w2_ref, b2_ref, o_ref):
    z = h_ref[...] * eps_ref[0, 0] + s_ref[...]
    a = jax.lax.dot(z, w1_ref[...], precision=lax.Precision.HIGHEST)
    a = jnp.maximum(a + b1_ref[...], 0.0)
    o = jax.lax.dot(a, w2_ref[...], precision=lax.Precision.HIGHEST)
    o_ref[...] = o + b2_ref[...]


def _mlp(h, s_agg, eps, w1, b1, w2, b2):
    eps11 = (1.0 + eps).reshape(1, 1)
    b1r = b1.reshape(1, D)
    b2r = b2.reshape(1, D)
    return pl.pallas_call(
        _mlp_body,
        grid=(N // BR,),
        in_specs=[
            pl.BlockSpec(memory_space=pltpu.SMEM),
            pl.BlockSpec((BR, D), lambda i: (i, 0)),
            pl.BlockSpec((BR, D), lambda i: (i, 0)),
            pl.BlockSpec((D, D), lambda i: (0, 0)),
            pl.BlockSpec((1, D), lambda i: (0, 0)),
            pl.BlockSpec((D, D), lambda i: (0, 0)),
            pl.BlockSpec((1, D), lambda i: (0, 0)),
        ],
        out_specs=pl.BlockSpec((BR, D), lambda i: (i, 0)),
        out_shape=jax.ShapeDtypeStruct((N, D), jnp.float32),
    )(eps11, h, s_agg, w1, b1r, w2, b2r)


def kernel(X, edge_index,
           eps_0, W1_0, b1_0, W2_0, b2_0,
           eps_1, W1_1, b1_1, W2_1, b2_1,
           eps_2, W1_2, b1_2, W2_2, b2_2):
    src = edge_index[0]
    dst = edge_index[1]
    pos = jnp.arange(E, dtype=jnp.int32)
    garb = HALF + (pos % NG)
    d0 = jnp.where(dst < HALF, dst, garb)
    d1 = jnp.where(dst >= HALF, dst - HALF, garb)
    dstl = jnp.concatenate([d0, d1])

    params = [
        (eps_0, W1_0, b1_0, W2_0, b2_0),
        (eps_1, W1_1, b1_1, W2_1, b2_1),
        (eps_2, W1_2, b1_2, W2_2, b2_2),
    ]
    h = X
    for (eps, w1, b1, w2, b2) in params:
        s_agg = _sc_segment_sum(h, src, dstl)
        h = _mlp(h, s_agg, eps, w1, b1, w2, b2)
    return h


# speedup vs baseline: 1.7666x; 1.7666x over previous
import dataclasses
import functools

import jax
import jax.numpy as jnp
from jax import lax
from jax.experimental import pallas as pl
from jax.experimental.pallas import tpu as pltpu
from jax.experimental.pallas import tpu_sc as plsc

N = 10000
D = 256
E = 160000

NC = 2
NS = 16
NW_ALL = NC * NS
RPT = 320
N_PAD = NW_ALL * RPT
W = 80
E_PAD = E + W

BR = 1024


def _sc_segment_sum(h, srcs, dsts, astart, nwin):
    mesh = plsc.VectorSubcoreMesh(core_axis_name="c", subcore_axis_name="s")
    cp = pltpu.CompilerParams()
    if "needs_layout_passes" in pltpu.CompilerParams.__dataclass_fields__:
        cp = dataclasses.replace(cp, needs_layout_passes=False)

    @functools.partial(
        pl.kernel,
        out_type=jax.ShapeDtypeStruct((N_PAD, D), jnp.float32),
        mesh=mesh,
        compiler_params=cp,
        scratch_types=[
            pltpu.VMEM((W,), jnp.int32),
            pltpu.VMEM((W,), jnp.int32),
            pltpu.VMEM((W,), jnp.int32),
            pltpu.VMEM((16,), jnp.int32),
            pltpu.VMEM((W, D), jnp.float32),
            pltpu.VMEM((RPT + 1, D), jnp.float32),
        ],
    )
    def seg_sum_kernel(h_hbm, src_hbm, dst_hbm, a_hbm, nw_hbm, s_hbm,
                       sidx, didx, dloc, tvec, stage, acc):
        c = lax.axis_index("c")
        s = lax.axis_index("s")
        t = c * NS + s
        base = t * RPT

        zeros16 = jnp.zeros((16,), jnp.float32)

        @pl.loop(0, RPT + 1)
        def _(r):
            for j in range(D // 16):
                acc[r, pl.ds(j * 16, 16)] = zeros16

        pltpu.sync_copy(a_hbm.at[t], tvec)
        a = pl.multiple_of(jnp.max(tvec[...]), 8)
        pltpu.sync_copy(nw_hbm.at[t], tvec)
        nw = jnp.max(tvec[...])

        basevec = jnp.full((16,), base, jnp.int32)
        garbvec = jnp.full((16,), RPT, jnp.int32)
        iota16 = lax.iota(jnp.int32, 16)

        @pl.loop(0, nw)
        def _(w):
            off = pl.multiple_of(a + w * W, 8)
            pltpu.sync_copy(src_hbm.at[pl.ds(off, W)], sidx)
            pltpu.sync_copy(dst_hbm.at[pl.ds(off, W)], didx)
            pltpu.sync_copy(h_hbm.at[sidx], stage)

            for q in range(W // 16):
                dvec = didx[pl.ds(q * 16, 16)]
                rel = dvec - basevec
                ok = (rel >= 0) & (rel < garbvec)
                dloc[pl.ds(q * 16, 16)] = jnp.where(ok, rel, garbvec)

            @pl.loop(0, W)
            def _(j):
                jvec = jnp.full((16,), j, jnp.int32)
                row = plsc.load_gather(dloc, [jvec])
                for m in range(D // 16):
                    cols = iota16 + (m * 16)
                    val = stage[j, pl.ds(m * 16, 16)]
                    plsc.addupdate_scatter(acc, [row, cols], val)

        pltpu.sync_copy(acc.at[pl.ds(0, RPT)], s_hbm.at[pl.ds(base, RPT)])

    return seg_sum_kernel(h, srcs, dsts, astart, nwin)


def _mlp_body(eps_ref, h_ref, s_ref, w1_ref, b1_ref, w2_ref, b2_ref, o_ref):
    z = h_ref[...] * eps_ref[0, 0] + s_ref[...]
    a = jax.lax.dot(z, w1_ref[...], precision=lax.Precision.HIGHEST)
    a = jnp.maximum(a + b1_ref[...], 0.0)
    o = jax.lax.dot(a, w2_ref[...], precision=lax.Precision.HIGHEST)
    o_ref[...] = o + b2_ref[...]


def _mlp(h, s_agg, eps, w1, b1, w2, b2):
    eps11 = (1.0 + eps).reshape(1, 1)
    b1r = b1.reshape(1, D)
    b2r = b2.reshape(1, D)
    return pl.pallas_call(
        _mlp_body,
        grid=(N_PAD // BR,),
        in_specs=[
            pl.BlockSpec(memory_space=pltpu.SMEM),
            pl.BlockSpec((BR, D), lambda i: (i, 0)),
            pl.BlockSpec((BR, D), lambda i: (i, 0)),
            pl.BlockSpec((D, D), lambda i: (0, 0)),
            pl.BlockSpec((1, D), lambda i: (0, 0)),
            pl.BlockSpec((D, D), lambda i: (0, 0)),
            pl.BlockSpec((1, D), lambda i: (0, 0)),
        ],
        out_specs=pl.BlockSpec((BR, D), lambda i: (i, 0)),
        out_shape=jax.ShapeDtypeStruct((N_PAD, D), jnp.float32),
    )(eps11, h, s_agg, w1, b1r, w2, b2r)


def kernel(X, edge_index,
           eps_0, W1_0, b1_0, W2_0, b2_0,
           eps_1, W1_1, b1_1, W2_1, b2_1,
           eps_2, W1_2, b1_2, W2_2, b2_2):
    src = edge_index[0]
    dst = edge_index[1]

    dst_s, src_s = lax.sort((dst, src), num_keys=1)
    dst_s = jnp.concatenate([dst_s, jnp.full((W,), jnp.int32(2**30))])
    src_s = jnp.concatenate([src_s, jnp.zeros((W,), jnp.int32)])

    bases = jnp.arange(NW_ALL + 1, dtype=jnp.int32) * RPT
    bounds = jnp.searchsorted(dst_s[:E], bases).astype(jnp.int32)
    starts, ends = bounds[:-1], bounds[1:]
    astart = (starts // 8) * 8
    nwin = (ends - astart + (W - 1)) // W
    a_b = jnp.broadcast_to(astart[:, None], (NW_ALL, 16)).astype(jnp.int32)
    nw_b = jnp.broadcast_to(nwin[:, None], (NW_ALL, 16)).astype(jnp.int32)

    params = [
        (eps_0, W1_0, b1_0, W2_0, b2_0),
        (eps_1, W1_1, b1_1, W2_1, b2_1),
        (eps_2, W1_2, b1_2, W2_2, b2_2),
    ]
    h = jnp.pad(X, ((0, N_PAD - N), (0, 0)))
    for (eps, w1, b1, w2, b2) in params:
        s_agg = _sc_segment_sum(h, src_s, dst_s, a_b, nw_b)
        h = _mlp(h, s_agg, eps, w1, b1, w2, b2)
    return h[:N]


# speedup vs baseline: 2.3270x; 1.3172x over previous
import dataclasses
import functools

import jax
import jax.numpy as jnp
from jax import lax
from jax.experimental import pallas as pl
from jax.experimental.pallas import tpu as pltpu
from jax.experimental.pallas import tpu_sc as plsc

N = 10000
D = 256
E = 160000

NC = 2
NS = 16
NW_ALL = NC * NS
RPT = 320
N_PAD = NW_ALL * RPT
W = 64
UE = 4

BR = 1024


def _sc_segment_sum(h, srcs, dsts, astart, nwin):
    mesh = plsc.VectorSubcoreMesh(core_axis_name="c", subcore_axis_name="s")
    cp = pltpu.CompilerParams()
    if "needs_layout_passes" in pltpu.CompilerParams.__dataclass_fields__:
        cp = dataclasses.replace(cp, needs_layout_passes=False)

    @functools.partial(
        pl.kernel,
        out_type=jax.ShapeDtypeStruct((N_PAD, D), jnp.float32),
        mesh=mesh,
        compiler_params=cp,
        scratch_types=[
            pltpu.VMEM((W,), jnp.int32),
            pltpu.VMEM((W,), jnp.int32),
            pltpu.VMEM((W,), jnp.int32),
            pltpu.VMEM((W,), jnp.int32),
            pltpu.VMEM((W,), jnp.int32),
            pltpu.VMEM((16,), jnp.int32),
            pltpu.VMEM((W, D), jnp.float32),
            pltpu.VMEM((W, D), jnp.float32),
            pltpu.VMEM((RPT + 1, D), jnp.float32),
            pltpu.SemaphoreType.DMA,
            pltpu.SemaphoreType.DMA,
            pltpu.SemaphoreType.DMA,
            pltpu.SemaphoreType.DMA,
            pltpu.SemaphoreType.DMA,
            pltpu.SemaphoreType.DMA,
        ],
    )
    def seg_sum_kernel(h_hbm, src_hbm, dst_hbm, a_hbm, nw_hbm, s_hbm,
                       sidx0, sidx1, didx0, didx1, dloc, tvec, stage0, stage1,
                       acc, ssem0, ssem1, dsem0, dsem1, gsem0, gsem1):
        c = lax.axis_index("c")
        s = lax.axis_index("s")
        t = c * NS + s
        base = t * RPT

        sidx = (sidx0, sidx1)
        didx = (didx0, didx1)
        stage = (stage0, stage1)
        ssem = (ssem0, ssem1)
        dsem = (dsem0, dsem1)
        gsem = (gsem0, gsem1)

        zeros16 = jnp.zeros((16,), jnp.float32)

        @pl.loop(0, RPT + 1)
        def _(r):
            for j in range(D // 16):
                acc[r, pl.ds(j * 16, 16)] = zeros16

        pltpu.sync_copy(a_hbm.at[t], tvec)
        a = pl.multiple_of(jnp.max(tvec[...]), 8)
        pltpu.sync_copy(nw_hbm.at[t], tvec)
        nw = jnp.max(tvec[...])

        basevec = jnp.full((16,), base, jnp.int32)
        garbvec = jnp.full((16,), RPT, jnp.int32)
        iota16 = lax.iota(jnp.int32, 16)

        def start_idx(w, b):
            off = pl.multiple_of(a + w * W, 8)
            pltpu.async_copy(src_hbm.at[pl.ds(off, W)], sidx[b], ssem[b])
            pltpu.async_copy(dst_hbm.at[pl.ds(off, W)], didx[b], dsem[b])

        def wait_idx(b):
            pltpu.make_async_copy(src_hbm.at[pl.ds(0, W)], sidx[b], ssem[b]).wait()
            pltpu.make_async_copy(dst_hbm.at[pl.ds(0, W)], didx[b], dsem[b]).wait()

        def start_gather(b):
            pltpu.async_copy(h_hbm.at[sidx[b]], stage[b], gsem[b])

        def wait_gather(b):
            pltpu.make_async_copy(h_hbm.at[sidx[b]], stage[b], gsem[b]).wait()

        def localize(b):
            for q in range(W // 16):
                dvec = didx[b][pl.ds(q * 16, 16)]
                rel = dvec - basevec
                ok = (rel >= 0) & (rel < garbvec)
                dloc[pl.ds(q * 16, 16)] = jnp.where(ok, rel, garbvec)

        def accumulate(b):
            @pl.loop(0, W // UE)
            def _(j0):
                for u in range(UE):
                    j = j0 * UE + u
                    jvec = jnp.full((16,), j, jnp.int32)
                    row = plsc.load_gather(dloc, [jvec])
                    for m in range(D // 16):
                        cols = iota16 + (m * 16)
                        val = stage[b][j, pl.ds(m * 16, 16)]
                        plsc.addupdate_scatter(acc, [row, cols], val)

        @pl.when(nw > 0)
        def _():
            start_idx(0, 0)

        @pl.when(nw > 1)
        def _():
            start_idx(1, 1)

        @pl.when(nw > 0)
        def _():
            wait_idx(0)
            start_gather(0)

        @pl.loop(0, nw, step=2)
        def _(w):
            @pl.when(w + 1 < nw)
            def _():
                wait_idx(1)
                start_gather(1)

            wait_gather(0)
            localize(0)

            @pl.when(w + 2 < nw)
            def _():
                start_idx(w + 2, 0)

            accumulate(0)

            @pl.when(w + 2 < nw)
            def _():
                wait_idx(0)
                start_gather(0)

            @pl.when(w + 1 < nw)
            def _():
                wait_gather(1)
                localize(1)

                @pl.when(w + 3 < nw)
                def _():
                    start_idx(w + 3, 1)

                accumulate(1)

        pltpu.sync_copy(acc.at[pl.ds(0, RPT)], s_hbm.at[pl.ds(base, RPT)])

    return seg_sum_kernel(h, srcs, dsts, astart, nwin)


def _mlp_body(eps_ref, h_ref, s_ref, w1_ref, b1_ref, w2_ref, b2_ref, o_ref):
    z = h_ref[...] * eps_ref[0, 0] + s_ref[...]
    a = jax.lax.dot(z, w1_ref[...], precision=lax.Precision.HIGHEST)
    a = jnp.maximum(a + b1_ref[...], 0.0)
    o = jax.lax.dot(a, w2_ref[...], precision=lax.Precision.HIGHEST)
    o_ref[...] = o + b2_ref[...]


def _mlp(h, s_agg, eps, w1, b1, w2, b2):
    eps11 = (1.0 + eps).reshape(1, 1)
    b1r = b1.reshape(1, D)
    b2r = b2.reshape(1, D)
    return pl.pallas_call(
        _mlp_body,
        grid=(N_PAD // BR,),
        in_specs=[
            pl.BlockSpec(memory_space=pltpu.SMEM),
            pl.BlockSpec((BR, D), lambda i: (i, 0)),
            pl.BlockSpec((BR, D), lambda i: (i, 0)),
            pl.BlockSpec((D, D), lambda i: (0, 0)),
            pl.BlockSpec((1, D), lambda i: (0, 0)),
            pl.BlockSpec((D, D), lambda i: (0, 0)),
            pl.BlockSpec((1, D), lambda i: (0, 0)),
        ],
        out_specs=pl.BlockSpec((BR, D), lambda i: (i, 0)),
        out_shape=jax.ShapeDtypeStruct((N_PAD, D), jnp.float32),
    )(eps11, h, s_agg, w1, b1r, w2, b2r)


def kernel(X, edge_index,
           eps_0, W1_0, b1_0, W2_0, b2_0,
           eps_1, W1_1, b1_1, W2_1, b2_1,
           eps_2, W1_2, b1_2, W2_2, b2_2):
    src = edge_index[0]
    dst = edge_index[1]

    dst_s, src_s = lax.sort((dst, src), num_keys=1)
    dst_s = jnp.concatenate([dst_s, jnp.full((W,), jnp.int32(2**30))])
    src_s = jnp.concatenate([src_s, jnp.zeros((W,), jnp.int32)])

    bases = jnp.arange(NW_ALL + 1, dtype=jnp.int32) * RPT
    bounds = jnp.searchsorted(dst_s[:E], bases).astype(jnp.int32)
    starts, ends = bounds[:-1], bounds[1:]
    astart = (starts // 8) * 8
    nwin = (ends - astart + (W - 1)) // W
    a_b = jnp.broadcast_to(astart[:, None], (NW_ALL, 16)).astype(jnp.int32)
    nw_b = jnp.broadcast_to(nwin[:, None], (NW_ALL, 16)).astype(jnp.int32)

    params = [
        (eps_0, W1_0, b1_0, W2_0, b2_0),
        (eps_1, W1_1, b1_1, W2_1, b2_1),
        (eps_2, W1_2, b1_2, W2_2, b2_2),
    ]
    h = jnp.pad(X, ((0, N_PAD - N), (0, 0)))
    for (eps, w1, b1, w2, b2) in params:
        s_agg = _sc_segment_sum(h, src_s, dst_s, a_b, nw_b)
        h = _mlp(h, s_agg, eps, w1, b1, w2, b2)
    return h[:N]


# speedup vs baseline: 4.3276x; 1.8597x over previous
import dataclasses
import functools

import jax
import jax.numpy as jnp
from jax import lax
from jax.experimental import pallas as pl
from jax.experimental.pallas import tpu as pltpu
from jax.experimental.pallas import tpu_sc as plsc

N = 10000
D = 256
E = 160000

NC = 2
NS = 16
NW_ALL = NC * NS
RPT = 320
N_PAD = NW_ALL * RPT
W = 64
UE = 4

BR = 1024


def _sc_segment_sum(h, srcs, dsts, astart, nwin):
    mesh = plsc.VectorSubcoreMesh(core_axis_name="c", subcore_axis_name="s")
    cp = pltpu.CompilerParams()
    if "needs_layout_passes" in pltpu.CompilerParams.__dataclass_fields__:
        cp = dataclasses.replace(cp, needs_layout_passes=False)

    @functools.partial(
        pl.kernel,
        out_type=jax.ShapeDtypeStruct((N_PAD, D), jnp.float32),
        mesh=mesh,
        compiler_params=cp,
        scratch_types=[
            pltpu.VMEM((W,), jnp.int32),
            pltpu.VMEM((W,), jnp.int32),
            pltpu.VMEM((W,), jnp.int32),
            pltpu.VMEM((W,), jnp.int32),
            pltpu.VMEM((W,), jnp.int32),
            pltpu.VMEM((16,), jnp.int32),
            pltpu.VMEM((W, D), jnp.float32),
            pltpu.VMEM((W, D), jnp.float32),
            pltpu.VMEM((RPT + 1, D), jnp.float32),
            pltpu.SemaphoreType.DMA,
            pltpu.SemaphoreType.DMA,
            pltpu.SemaphoreType.DMA,
            pltpu.SemaphoreType.DMA,
            pltpu.SemaphoreType.DMA,
            pltpu.SemaphoreType.DMA,
        ],
    )
    def seg_sum_kernel(h_hbm, src_hbm, dst_hbm, a_hbm, nw_hbm, s_hbm,
                       sidx0, sidx1, didx0, didx1, dloc, tvec, stage0, stage1,
                       acc, ssem0, ssem1, dsem0, dsem1, gsem0, gsem1):
        c = lax.axis_index("c")
        s = lax.axis_index("s")
        t = c * NS + s
        base = t * RPT

        sidx = (sidx0, sidx1)
        didx = (didx0, didx1)
        stage = (stage0, stage1)
        ssem = (ssem0, ssem1)
        dsem = (dsem0, dsem1)
        gsem = (gsem0, gsem1)

        zeros16 = jnp.zeros((16,), jnp.float32)

        @pl.loop(0, RPT + 1)
        def _(r):
            for j in range(D // 16):
                acc[r, pl.ds(j * 16, 16)] = zeros16

        pltpu.sync_copy(a_hbm.at[t], tvec)
        a = pl.multiple_of(jnp.max(tvec[...]), 8)
        pltpu.sync_copy(nw_hbm.at[t], tvec)
        nw = jnp.max(tvec[...])

        basevec = jnp.full((16,), base, jnp.int32)
        garbvec = jnp.full((16,), RPT, jnp.int32)
        iota16 = lax.iota(jnp.int32, 16)

        def start_idx(w, b):
            off = pl.multiple_of(a + w * W, 8)
            pltpu.async_copy(src_hbm.at[pl.ds(off, W)], sidx[b], ssem[b])
            pltpu.async_copy(dst_hbm.at[pl.ds(off, W)], didx[b], dsem[b])

        def wait_idx(b):
            pltpu.make_async_copy(src_hbm.at[pl.ds(0, W)], sidx[b], ssem[b]).wait()
            pltpu.make_async_copy(dst_hbm.at[pl.ds(0, W)], didx[b], dsem[b]).wait()

        def start_gather(b):
            pltpu.async_copy(h_hbm.at[sidx[b]], stage[b], gsem[b])

        def wait_gather(b):
            pltpu.make_async_copy(h_hbm.at[sidx[b]], stage[b], gsem[b]).wait()

        def localize(b):
            for q in range(W // 16):
                dvec = didx[b][pl.ds(q * 16, 16)]
                rel = dvec - basevec
                ok = (rel >= 0) & (rel < garbvec)
                dloc[pl.ds(q * 16, 16)] = jnp.where(ok, rel, garbvec)

        def accumulate(b):
            @plsc.parallel_loop(0, W, step=1, unroll=UE)
            def _(j):
                jvec = jnp.full((16,), j, jnp.int32)
                row = plsc.load_gather(dloc, [jvec])
                for m in range(D // 16):
                    cols = iota16 + (m * 16)
                    val = stage[b][j, pl.ds(m * 16, 16)]
                    plsc.addupdate_scatter(acc, [row, cols], val)

        @pl.when(nw > 0)
        def _():
            start_idx(0, 0)

        @pl.when(nw > 1)
        def _():
            start_idx(1, 1)

        @pl.when(nw > 0)
        def _():
            wait_idx(0)
            start_gather(0)

        @pl.loop(0, nw, step=2)
        def _(w):
            @pl.when(w + 1 < nw)
            def _():
                wait_idx(1)
                start_gather(1)

            wait_gather(0)
            localize(0)

            @pl.when(w + 2 < nw)
            def _():
                start_idx(w + 2, 0)

            accumulate(0)

            @pl.when(w + 2 < nw)
            def _():
                wait_idx(0)
                start_gather(0)

            @pl.when(w + 1 < nw)
            def _():
                wait_gather(1)
                localize(1)

                @pl.when(w + 3 < nw)
                def _():
                    start_idx(w + 3, 1)

                accumulate(1)

        pltpu.sync_copy(acc.at[pl.ds(0, RPT)], s_hbm.at[pl.ds(base, RPT)])

    return seg_sum_kernel(h, srcs, dsts, astart, nwin)


def _mlp_body(eps_ref, h_ref, s_ref, w1_ref, b1_ref, w2_ref, b2_ref, o_ref):
    z = h_ref[...] * eps_ref[0, 0] + s_ref[...]
    a = jax.lax.dot(z, w1_ref[...], precision=lax.Precision.HIGHEST)
    a = jnp.maximum(a + b1_ref[...], 0.0)
    o = jax.lax.dot(a, w2_ref[...], precision=lax.Precision.HIGHEST)
    o_ref[...] = o + b2_ref[...]


def _mlp(h, s_agg, eps, w1, b1, w2, b2):
    eps11 = (1.0 + eps).reshape(1, 1)
    b1r = b1.reshape(1, D)
    b2r = b2.reshape(1, D)
    return pl.pallas_call(
        _mlp_body,
        grid=(N_PAD // BR,),
        in_specs=[
            pl.BlockSpec(memory_space=pltpu.SMEM),
            pl.BlockSpec((BR, D), lambda i: (i, 0)),
            pl.BlockSpec((BR, D), lambda i: (i, 0)),
            pl.BlockSpec((D, D), lambda i: (0, 0)),
            pl.BlockSpec((1, D), lambda i: (0, 0)),
            pl.BlockSpec((D, D), lambda i: (0, 0)),
            pl.BlockSpec((1, D), lambda i: (0, 0)),
        ],
        out_specs=pl.BlockSpec((BR, D), lambda i: (i, 0)),
        out_shape=jax.ShapeDtypeStruct((N_PAD, D), jnp.float32),
    )(eps11, h, s_agg, w1, b1r, w2, b2r)


def kernel(X, edge_index,
           eps_0, W1_0, b1_0, W2_0, b2_0,
           eps_1, W1_1, b1_1, W2_1, b2_1,
           eps_2, W1_2, b1_2, W2_2, b2_2):
    src = edge_index[0]
    dst = edge_index[1]

    dst_s, src_s = lax.sort((dst, src), num_keys=1)
    dst_s = jnp.concatenate([dst_s, jnp.full((W,), jnp.int32(2**30))])
    src_s = jnp.concatenate([src_s, jnp.zeros((W,), jnp.int32)])

    bases = jnp.arange(NW_ALL + 1, dtype=jnp.int32) * RPT
    bounds = jnp.searchsorted(dst_s[:E], bases).astype(jnp.int32)
    starts, ends = bounds[:-1], bounds[1:]
    astart = (starts // 8) * 8
    nwin = (ends - astart + (W - 1)) // W
    a_b = jnp.broadcast_to(astart[:, None], (NW_ALL, 16)).astype(jnp.int32)
    nw_b = jnp.broadcast_to(nwin[:, None], (NW_ALL, 16)).astype(jnp.int32)

    params = [
        (eps_0, W1_0, b1_0, W2_0, b2_0),
        (eps_1, W1_1, b1_1, W2_1, b2_1),
        (eps_2, W1_2, b1_2, W2_2, b2_2),
    ]
    h = jnp.pad(X, ((0, N_PAD - N), (0, 0)))
    for (eps, w1, b1, w2, b2) in params:
        s_agg = _sc_segment_sum(h, src_s, dst_s, a_b, nw_b)
        h = _mlp(h, s_agg, eps, w1, b1, w2, b2)
    return h[:N]


# speedup vs baseline: 4.8117x; 1.1119x over previous
import dataclasses
import functools

import jax
import jax.numpy as jnp
from jax import lax
from jax.experimental import pallas as pl
from jax.experimental.pallas import tpu as pltpu
from jax.experimental.pallas import tpu_sc as plsc

N = 10000
D = 256
E = 160000

NC = 2
NS = 16
NW_ALL = NC * NS
RPT = 320
N_PAD = NW_ALL * RPT
W = 80
UE = 4

BR = 1024


def _sc_segment_sum(h, srcs, dsts, astart, nwin):
    mesh = plsc.VectorSubcoreMesh(core_axis_name="c", subcore_axis_name="s")
    cp = pltpu.CompilerParams()
    if "needs_layout_passes" in pltpu.CompilerParams.__dataclass_fields__:
        cp = dataclasses.replace(cp, needs_layout_passes=False)

    @functools.partial(
        pl.kernel,
        out_type=jax.ShapeDtypeStruct((N_PAD, D), jnp.float32),
        mesh=mesh,
        compiler_params=cp,
        scratch_types=[
            pltpu.VMEM((W,), jnp.int32),
            pltpu.VMEM((W,), jnp.int32),
            pltpu.VMEM((W,), jnp.int32),
            pltpu.VMEM((W,), jnp.int32),
            pltpu.VMEM((W,), jnp.int32),
            pltpu.VMEM((16,), jnp.int32),
            pltpu.VMEM((W, D), jnp.float32),
            pltpu.VMEM((W, D), jnp.float32),
            pltpu.VMEM((RPT + 1, D), jnp.float32),
            pltpu.SemaphoreType.DMA,
            pltpu.SemaphoreType.DMA,
            pltpu.SemaphoreType.DMA,
            pltpu.SemaphoreType.DMA,
            pltpu.SemaphoreType.DMA,
            pltpu.SemaphoreType.DMA,
        ],
    )
    def seg_sum_kernel(h_hbm, src_hbm, dst_hbm, a_hbm, nw_hbm, s_hbm,
                       sidx0, sidx1, didx0, didx1, dloc, tvec, stage0, stage1,
                       acc, ssem0, ssem1, dsem0, dsem1, gsem0, gsem1):
        c = lax.axis_index("c")
        s = lax.axis_index("s")
        t = c * NS + s
        base = t * RPT

        sidx = (sidx0, sidx1)
        didx = (didx0, didx1)
        stage = (stage0, stage1)
        ssem = (ssem0, ssem1)
        dsem = (dsem0, dsem1)
        gsem = (gsem0, gsem1)

        zeros16 = jnp.zeros((16,), jnp.float32)

        @pl.loop(0, RPT + 1)
        def _(r):
            for j in range(D // 16):
                acc[r, pl.ds(j * 16, 16)] = zeros16

        pltpu.sync_copy(a_hbm.at[t], tvec)
        a = pl.multiple_of(jnp.max(tvec[...]), 8)
        pltpu.sync_copy(nw_hbm.at[t], tvec)
        nw = jnp.max(tvec[...])

        basevec = jnp.full((16,), base, jnp.int32)
        garbvec = jnp.full((16,), RPT, jnp.int32)
        iota16 = lax.iota(jnp.int32, 16)

        def start_idx(w, b):
            off = pl.multiple_of(a + w * W, 8)
            pltpu.async_copy(src_hbm.at[pl.ds(off, W)], sidx[b], ssem[b])
            pltpu.async_copy(dst_hbm.at[pl.ds(off, W)], didx[b], dsem[b])

        def wait_idx(b):
            pltpu.make_async_copy(src_hbm.at[pl.ds(0, W)], sidx[b], ssem[b]).wait()
            pltpu.make_async_copy(dst_hbm.at[pl.ds(0, W)], didx[b], dsem[b]).wait()

        def start_gather(b):
            pltpu.async_copy(h_hbm.at[sidx[b]], stage[b], gsem[b])

        def wait_gather(b):
            pltpu.make_async_copy(h_hbm.at[sidx[b]], stage[b], gsem[b]).wait()

        def localize(b):
            for q in range(W // 16):
                dvec = didx[b][pl.ds(q * 16, 16)]
                rel = dvec - basevec
                ok = (rel >= 0) & (rel < garbvec)
                dloc[pl.ds(q * 16, 16)] = jnp.where(ok, rel, garbvec)

        def accumulate(b):
            @plsc.parallel_loop(0, W, step=1, unroll=UE)
            def _(j):
                jvec = jnp.full((16,), j, jnp.int32)
                row = plsc.load_gather(dloc, [jvec])
                for m in range(D // 16):
                    cols = iota16 + (m * 16)
                    val = stage[b][j, pl.ds(m * 16, 16)]
                    plsc.addupdate_scatter(acc, [row, cols], val)

        @pl.when(nw > 0)
        def _():
            start_idx(0, 0)

        @pl.when(nw > 1)
        def _():
            start_idx(1, 1)

        @pl.when(nw > 0)
        def _():
            wait_idx(0)
            start_gather(0)

        @pl.loop(0, nw, step=2)
        def _(w):
            @pl.when(w + 1 < nw)
            def _():
                wait_idx(1)
                start_gather(1)

            wait_gather(0)
            localize(0)

            @pl.when(w + 2 < nw)
            def _():
                start_idx(w + 2, 0)

            accumulate(0)

            @pl.when(w + 2 < nw)
            def _():
                wait_idx(0)
                start_gather(0)

            @pl.when(w + 1 < nw)
            def _():
                wait_gather(1)
                localize(1)

                @pl.when(w + 3 < nw)
                def _():
                    start_idx(w + 3, 1)

                accumulate(1)

        pltpu.sync_copy(acc.at[pl.ds(0, RPT)], s_hbm.at[pl.ds(base, RPT)])

    return seg_sum_kernel(h, srcs, dsts, astart, nwin)


def _mlp_body(eps_ref, h_ref, s_ref, w1_ref, b1_ref, w2_ref, b2_ref, o_ref):
    z = h_ref[...] * eps_ref[0, 0] + s_ref[...]
    a = jax.lax.dot(z, w1_ref[...])
    a = jnp.maximum(a + b1_ref[...], 0.0)
    o = jax.lax.dot(a, w2_ref[...])
    o_ref[...] = o + b2_ref[...]


def _mlp(h, s_agg, eps, w1, b1, w2, b2):
    eps11 = (1.0 + eps).reshape(1, 1)
    b1r = b1.reshape(1, D)
    b2r = b2.reshape(1, D)
    return pl.pallas_call(
        _mlp_body,
        grid=(N_PAD // BR,),
        in_specs=[
            pl.BlockSpec(memory_space=pltpu.SMEM),
            pl.BlockSpec((BR, D), lambda i: (i, 0)),
            pl.BlockSpec((BR, D), lambda i: (i, 0)),
            pl.BlockSpec((D, D), lambda i: (0, 0)),
            pl.BlockSpec((1, D), lambda i: (0, 0)),
            pl.BlockSpec((D, D), lambda i: (0, 0)),
            pl.BlockSpec((1, D), lambda i: (0, 0)),
        ],
        out_specs=pl.BlockSpec((BR, D), lambda i: (i, 0)),
        out_shape=jax.ShapeDtypeStruct((N_PAD, D), jnp.float32),
    )(eps11, h, s_agg, w1, b1r, w2, b2r)


def kernel(X, edge_index,
           eps_0, W1_0, b1_0, W2_0, b2_0,
           eps_1, W1_1, b1_1, W2_1, b2_1,
           eps_2, W1_2, b1_2, W2_2, b2_2):
    src = edge_index[0]
    dst = edge_index[1]

    comb = lax.sort(jnp.left_shift(dst, 14) | src)
    dst_s = jnp.right_shift(comb, 14)
    src_s = comb & jnp.int32(16383)
    dst_s = jnp.concatenate([dst_s, jnp.full((W,), jnp.int32(2**30))])
    src_s = jnp.concatenate([src_s, jnp.zeros((W,), jnp.int32)])

    bases = jnp.arange(NW_ALL + 1, dtype=jnp.int32) * RPT
    bounds = jnp.searchsorted(dst_s[:E], bases).astype(jnp.int32)
    starts, ends = bounds[:-1], bounds[1:]
    astart = (starts // 8) * 8
    nwin = (ends - astart + (W - 1)) // W
    a_b = jnp.broadcast_to(astart[:, None], (NW_ALL, 16)).astype(jnp.int32)
    nw_b = jnp.broadcast_to(nwin[:, None], (NW_ALL, 16)).astype(jnp.int32)

    params = [
        (eps_0, W1_0, b1_0, W2_0, b2_0),
        (eps_1, W1_1, b1_1, W2_1, b2_1),
        (eps_2, W1_2, b1_2, W2_2, b2_2),
    ]
    h = jnp.pad(X, ((0, N_PAD - N), (0, 0)))
    for (eps, w1, b1, w2, b2) in params:
        s_agg = _sc_segment_sum(h, src_s, dst_s, a_b, nw_b)
        h = _mlp(h, s_agg, eps, w1, b1, w2, b2)
    return h[:N]
